# Initial kernel scaffold; baseline (speedup 1.0000x reference)
import jax
import jax.numpy as jnp
from jax.experimental import pallas as pl


def kernel(forest, adjacency, node_order, edge_order, W, b):
    raise NotImplementedError("write your pallas kernel here")


# speedup vs baseline: 1.1918x; 1.1918x over previous
import jax
import jax.numpy as jnp
from jax import lax
from jax.experimental import pallas as pl

HIDDEN = 500
N_NODES = 31


def _fused_kernel(x_ref, w_ref, b_ref, no_ref, out_ref):
    x = x_ref[...]
    w = w_ref[...]
    b = b_ref[...]
    no = no_ref[...]

    acc = lax.dot_general(
        x, w,
        dimension_numbers=(((1,), (1,)), ((), ())),
        preferred_element_type=jnp.float32,
    )

    rows, cols = acc.shape
    h_idx = lax.broadcasted_iota(jnp.int32, (rows, cols), 1)
    r_idx = lax.broadcasted_iota(jnp.int32, (rows, cols), 0)
    n = r_idx % N_NODES
    d = h_idx // 3
    max_order = jnp.max(no)
    pe_mask = (
        (h_idx < 15)
        & (no == d)
        & (d < max_order)
        & (n != 0)
        & (h_idx % 3 == (n - 1) % 3)
    )
    out_ref[...] = acc + b + pe_mask.astype(jnp.float32)


def kernel(forest, adjacency, node_order, edge_order, W, b):
    batch, n_agents, n_nodes, feat = forest.shape
    rows = batch * n_agents * n_nodes
    x = forest.reshape(rows, feat)
    no = node_order.astype(jnp.int32).reshape(rows, 1)
    b2 = b.reshape(1, HIDDEN)

    out = pl.pallas_call(
        _fused_kernel,
        out_shape=jax.ShapeDtypeStruct((rows, HIDDEN), jnp.float32),
    )(x, W, b2, no)
    return out.reshape(batch, n_agents, n_nodes, HIDDEN)
